# Initial kernel scaffold; baseline (speedup 1.0000x reference)
import jax
import jax.numpy as jnp
from jax.experimental import pallas as pl


def kernel(X, g1_src, g1_dst, X0, alpha, beta, degE, degV, W):
    raise NotImplementedError("write your pallas kernel here")


# speedup vs baseline: 8.8907x; 8.8907x over previous
import functools

import jax
import jax.numpy as jnp
from jax import lax
from jax.experimental import pallas as pl
from jax.experimental.pallas import tpu as pltpu
from jax.experimental.pallas import tpu_sc as plsc

NC = 2
NS = 16
LANES = 16
K = 125


def _sc_two_pass(xh, srcs, dsts, dege_pad):
    _, n_pad, half = xh.shape
    _, ch, _ = srcs.shape
    nh_pad = dege_pad.shape[0]
    rpt_v = n_pad // NS
    rpt_e = nh_pad // NS
    nlv = half // LANES

    mesh = plsc.VectorSubcoreMesh(core_axis_name="c", subcore_axis_name="s")

    ZB = 80
    CPB = 40
    nblk = ch // CPB
    assert rpt_v % ZB == 0 and rpt_e % ZB == 0 and ch % CPB == 0

    @functools.partial(
        pl.kernel,
        out_type=jax.ShapeDtypeStruct((NC, n_pad, half), jnp.float32),
        mesh=mesh,
        compiler_params=pltpu.CompilerParams(use_tc_tiling_on_sc=False),
        scratch_types=[
            pltpu.VMEM_SHARED((nh_pad, half), jnp.float32),
            pltpu.VMEM_SHARED((n_pad, half), jnp.float32),
            pltpu.VMEM((ZB, half), jnp.float32),
            pltpu.VMEM((CPB, K), jnp.int32),
            pltpu.VMEM((CPB, K), jnp.int32),
            pltpu.VMEM((K, half), jnp.float32),
            pltpu.VMEM((K, half), jnp.float32),
            pltpu.VMEM((rpt_e,), jnp.float32),
            pltpu.SemaphoreType.DMA,
            pltpu.SemaphoreType.DMA,
            pltpu.SemaphoreType.DMA,
            pltpu.SemaphoreType.DMA,
        ],
    )
    def sc_kernel(xh_hbm, srcs_hbm, dsts_hbm, dege_hbm, out_hbm,
                  xesh, xvsh, zbuf, src_v, dst_v, rows0, rows1, dege_v,
                  semg0, semg1, sems0, sems1):
        cid = lax.axis_index("c")
        sid = lax.axis_index("s")

        def zrow(i, _):
            for l in range(nlv):
                zbuf[i, pl.ds(l * LANES, LANES)] = jnp.zeros((LANES,), jnp.float32)
            return 0
        lax.fori_loop(0, ZB, zrow, 0)

        def ze(b, _):
            pltpu.sync_copy(zbuf, xesh.at[pl.ds(sid * rpt_e + b * ZB, ZB)])
            return 0
        lax.fori_loop(0, rpt_e // ZB, ze, 0)

        def zv(b, _):
            pltpu.sync_copy(zbuf, xvsh.at[pl.ds(sid * rpt_v + b * ZB, ZB)])
            return 0
        lax.fori_loop(0, rpt_v // ZB, zv, 0)

        pltpu.sync_copy(dege_hbm.at[pl.ds(sid * rpt_e, rpt_e)], dege_v)

        plsc.subcore_barrier()

        def run_pass(gsrc, acc, idx_s):
            def blk(b, _):
                pltpu.sync_copy(srcs_hbm.at[sid, pl.ds(b * CPB, CPB)], src_v)
                pltpu.sync_copy(dsts_hbm.at[sid, pl.ds(b * CPB, CPB)], dst_v)
                pltpu.async_copy(gsrc(0), rows0, semg0)

                def step(j2, _):
                    j0 = 2 * j2
                    j1 = j0 + 1
                    pltpu.make_async_copy(gsrc(j0), rows0, semg0).wait()

                    @pl.when(j2 > 0)
                    def _():
                        pltpu.make_async_copy(
                            rows1, acc.at[idx_s.at[j1]], sems1).wait()
                    pltpu.async_copy(gsrc(j1), rows1, semg1)
                    pltpu.async_copy(rows0, acc.at[idx_s.at[j0]], sems0,
                                     add=True)
                    pltpu.make_async_copy(gsrc(j1), rows1, semg1).wait()
                    pltpu.async_copy(rows1, acc.at[idx_s.at[j1]], sems1,
                                     add=True)
                    pltpu.make_async_copy(
                        rows0, acc.at[idx_s.at[j0]], sems0).wait()

                    @pl.when(j2 < CPB // 2 - 1)
                    def _():
                        pltpu.async_copy(gsrc(j0 + 2), rows0, semg0)
                    return 0
                lax.fori_loop(0, CPB // 2, step, 0)
                pltpu.make_async_copy(
                    rows1, acc.at[idx_s.at[CPB - 1]], sems1).wait()
                return 0
            lax.fori_loop(0, nblk, blk, 0)

        run_pass(lambda j: xh_hbm.at[cid].at[src_v.at[j]], xesh, dst_v)

        plsc.subcore_barrier()

        def sblk(b, _):
            base = sid * rpt_e + b * ZB
            pltpu.sync_copy(xesh.at[pl.ds(base, ZB)], zbuf)

            def scale(g, _):
                dvec = dege_v[pl.ds(b * ZB + g * LANES, LANES)]
                for r in range(LANES):
                    sc = dvec[r]
                    row = g * LANES + r
                    for l in range(nlv):
                        zbuf[row, pl.ds(l * LANES, LANES)] = (
                            zbuf[row, pl.ds(l * LANES, LANES)] * sc)
                return 0
            lax.fori_loop(0, ZB // LANES, scale, 0)
            pltpu.sync_copy(zbuf, xesh.at[pl.ds(base, ZB)])
            return 0
        lax.fori_loop(0, rpt_e // ZB, sblk, 0)

        plsc.subcore_barrier()

        run_pass(lambda j: xesh.at[dst_v.at[j]], xvsh, src_v)

        plsc.subcore_barrier()

        def oblk(b, _):
            pltpu.sync_copy(xvsh.at[pl.ds(sid * rpt_v + b * ZB, ZB)], zbuf)
            pltpu.sync_copy(zbuf, out_hbm.at[cid, pl.ds(sid * rpt_v + b * ZB, ZB)])
            return 0
        lax.fori_loop(0, rpt_v // ZB, oblk, 0)

    return sc_kernel(xh, srcs, dsts, dege_pad)


def _tc_finish(xv2, x0, degv, w, alpha, beta):
    n, d = x0.shape
    half = d // NC
    bm = 1000
    grid = (n // bm,)

    def body(a_ref, b_ref, xv_ref, x0_ref, degv_ref, w_ref, o_ref):
        a = a_ref[0]
        b = b_ref[0]
        xv = jnp.concatenate([xv_ref[0], xv_ref[1]], axis=-1)
        xv = xv * degv_ref[...]
        xi = (1.0 - a) * xv + a * x0_ref[...]
        o_ref[...] = (1.0 - b) * xi + b * lax.dot_general(
            xi, w_ref[...], (((1,), (1,)), ((), ())),
            preferred_element_type=jnp.float32)

    return pl.pallas_call(
        body,
        grid=grid,
        in_specs=[
            pl.BlockSpec(memory_space=pltpu.SMEM),
            pl.BlockSpec(memory_space=pltpu.SMEM),
            pl.BlockSpec((NC, bm, half), lambda i: (0, i, 0)),
            pl.BlockSpec((bm, d), lambda i: (i, 0)),
            pl.BlockSpec((bm, 1), lambda i: (i, 0)),
            pl.BlockSpec((d, d), lambda i: (0, 0)),
        ],
        out_specs=pl.BlockSpec((bm, d), lambda i: (i, 0)),
        out_shape=jax.ShapeDtypeStruct((n, d), jnp.float32),
    )(alpha.reshape(1), beta.reshape(1), xv2, x0, degv, w)


def kernel(X, g1_src, g1_dst, X0, alpha, beta, degE, degV, W):
    n, d = X.shape
    nh = degE.shape[0]
    ninc = g1_src.shape[0]
    half = d // NC
    per_tile = ninc // NS
    ch = per_tile // K

    n_pad = ((n + NS * 80 - 1) // (NS * 80)) * (NS * 80)
    xh = X.reshape(n, NC, half).transpose(1, 0, 2)
    xh = jnp.pad(xh, ((0, 0), (0, n_pad - n), (0, 0)))
    srcs = g1_src.reshape(NS, ch, K)
    dsts = g1_dst.reshape(NS, ch, K)
    nh_pad = ((nh + NS * 80 - 1) // (NS * 80)) * (NS * 80)
    dege_pad = jnp.concatenate(
        [degE.reshape(-1), jnp.ones((nh_pad - nh,), jnp.float32)])

    xv2 = _sc_two_pass(xh, srcs, dsts, dege_pad)
    return _tc_finish(xv2, X0, degV, W, alpha, beta)
